# Initial kernel scaffold; baseline (speedup 1.0000x reference)
import jax
import jax.numpy as jnp
from jax.experimental import pallas as pl


def kernel(src_tokens, trg_tokens, emb, w, b):
    raise NotImplementedError("write your pallas kernel here")


# speedup vs baseline: 36.0235x; 36.0235x over previous
import functools

import jax
import jax.numpy as jnp
from jax import lax
from jax.experimental import pallas as pl
from jax.experimental.pallas import tpu as pltpu
from jax.experimental.pallas import tpu_sc as plsc

_PAD = 1
_V = 100000
_D = 128
_B = 4096
_L = 200
_CH = 64
_NCH = 4
_NG = _CH // 16
_MV_BLK = 2000


def _matvec_body(emb_ref, wr_ref, out_ref):
    out_ref[...] = jnp.dot(emb_ref[...], wr_ref[...],
                           preferred_element_type=jnp.float32)


def _tc_matvec(emb, wr):
    return pl.pallas_call(
        _matvec_body,
        grid=(_V // _MV_BLK,),
        in_specs=[
            pl.BlockSpec((_MV_BLK, _D), lambda i: (i, 0)),
            pl.BlockSpec((_D, 2), lambda i: (0, 0)),
        ],
        out_specs=pl.BlockSpec((_MV_BLK, 2), lambda i: (i, 0)),
        out_shape=jax.ShapeDtypeStruct((_V, 2), jnp.float32),
    )(emb, wr)


def _make_sc_pool():
    mesh = plsc.VectorSubcoreMesh(core_axis_name="c", subcore_axis_name="s")

    @functools.partial(
        pl.kernel,
        mesh=mesh,
        compiler_params=pltpu.CompilerParams(needs_layout_passes=False),
        out_type=jax.ShapeDtypeStruct((2, _B), jnp.float32),
        scratch_types=[
            pltpu.VMEM((_V,), jnp.float32),
            pltpu.VMEM((_L, _CH), jnp.int32),
            pltpu.VMEM((_CH,), jnp.float32),
        ],
    )
    def sc_pool(tables_hbm, toks_hbm, out_hbm, table_v, tok_v, part_v):
        c = lax.axis_index("c")
        s = lax.axis_index("s")
        pltpu.sync_copy(tables_hbm.at[c], table_v)
        for j in range(_NCH):
            k = s * _NCH + j
            pltpu.sync_copy(toks_hbm.at[c, k], tok_v)
            for g in range(_NG):
                def body(l, carry, g=g):
                    acc, cnt = carry
                    tok = tok_v[l, pl.ds(g * 16, 16)]
                    val = plsc.load_gather(table_v, [tok])
                    m = tok != _PAD
                    zero = jnp.zeros((16,), jnp.float32)
                    return (acc + jnp.where(m, val, zero),
                            cnt + jnp.where(m, jnp.ones((16,), jnp.float32),
                                            zero))
                acc, cnt = lax.fori_loop(
                    0, _L, body,
                    (jnp.zeros((16,), jnp.float32),
                     jnp.zeros((16,), jnp.float32)))
                part_v[pl.ds(g * 16, 16)] = acc / cnt
            pltpu.sync_copy(part_v, out_hbm.at[c, pl.ds(k * _CH, _CH)])

    return sc_pool


_sc_pool = _make_sc_pool()


def _combine_body(parts_ref, b_ref, out_ref):
    x = parts_ref[0, :] + parts_ref[1, :] + b_ref[0]
    out_ref[...] = 1.0 / (1.0 + jnp.exp(-x))


def _tc_combine(parts, b):
    return pl.pallas_call(
        _combine_body,
        in_specs=[
            pl.BlockSpec((2, _B), lambda: (0, 0)),
            pl.BlockSpec(memory_space=pltpu.SMEM),
        ],
        out_shape=jax.ShapeDtypeStruct((_B,), jnp.float32),
    )(parts, b)


def kernel(src_tokens, trg_tokens, emb, w, b):
    wr = jnp.concatenate([w[:_D], w[_D:]], axis=1)
    pq = _tc_matvec(emb, wr)
    tables = pq.T

    def prep(t):
        return t.astype(jnp.int32).reshape(
            _B // _CH, _CH, _L).transpose(0, 2, 1)

    toks = jnp.stack([prep(src_tokens), prep(trg_tokens)])
    parts = _sc_pool(tables, toks)
    score = _tc_combine(parts, b)
    return score.reshape(_B, 1)
